# baseline (speedup 1.0000x reference)
import functools

import jax
import jax.numpy as jnp
from jax import lax
from jax.experimental import pallas as pl
from jax.experimental.pallas import tpu as pltpu
from jax.experimental.pallas import tpu_sc as plsc

BATCH = 16384
LATENT_DIM = 32
HIDDEN = 16
LANES = 16

_info = plsc.get_sparse_core_info()
_NC, _NS = _info.num_cores, _info.num_subcores
_NW = _NC * _NS
_BPW = BATCH // _NW
_GROUPS = _BPW // LANES
_IDXW = 128
_NIDX = _BPW // _IDXW

_mesh = plsc.VectorSubcoreMesh(core_axis_name="c", subcore_axis_name="s")


@functools.partial(
    pl.kernel,
    mesh=_mesh,
    out_type=jax.ShapeDtypeStruct((BATCH,), jnp.float32),
    compiler_params=pltpu.CompilerParams(needs_layout_passes=False),
    scratch_types=[
        pltpu.VMEM((_NIDX, _IDXW), jnp.int32),
        pltpu.VMEM((_NIDX, _IDXW), jnp.int32),
        pltpu.VMEM((LATENT_DIM * _NIDX, _IDXW), jnp.int32),
        pltpu.VMEM((LATENT_DIM * _NIDX, _IDXW), jnp.int32),
        pltpu.VMEM((LATENT_DIM, _BPW), jnp.float32),
        pltpu.VMEM((LATENT_DIM, _BPW), jnp.float32),
        pltpu.VMEM((HIDDEN * LATENT_DIM,), jnp.float32),
        pltpu.VMEM((HIDDEN,), jnp.float32),
        pltpu.SMEM((HIDDEN * LATENT_DIM,), jnp.float32),
        pltpu.SMEM((HIDDEN,), jnp.float32),
        pltpu.VMEM((_BPW,), jnp.float32),
        pltpu.SemaphoreType.DMA,
        pltpu.SemaphoreType.DMA,
    ],
)
def _mf_sc(uids_hbm, iids_hbm, utab_hbm, itab_hbm, w_hbm, b_hbm, out_hbm,
           uidx_v, iidx_v, uwidx_v, iwidx_v, ubuf, ibuf, w_v, b_v, w_s, b_s,
           out_v, sem_u, sem_i):
    wid = lax.axis_index("s") * _NC + lax.axis_index("c")
    base = wid * _BPW

    for j in range(_NIDX):
        pltpu.sync_copy(uids_hbm.at[pl.ds(base + j * _IDXW, _IDXW)],
                        uidx_v.at[j])
        pltpu.sync_copy(iids_hbm.at[pl.ds(base + j * _IDXW, _IDXW)],
                        iidx_v.at[j])
    pltpu.sync_copy(w_hbm, w_v)
    pltpu.sync_copy(b_hbm, b_v)

    n_rows = utab_hbm.shape[0] // LATENT_DIM
    for j in range(_NIDX):
        for c in range(_IDXW // LANES):
            uid = uidx_v[j, pl.ds(c * LANES, LANES)]
            iid = iidx_v[j, pl.ds(c * LANES, LANES)]
            for d in range(LATENT_DIM):
                uwidx_v[d * _NIDX + j, pl.ds(c * LANES, LANES)] = (
                    uid + d * n_rows)
                iwidx_v[d * _NIDX + j, pl.ds(c * LANES, LANES)] = (
                    iid + d * n_rows)

    copies = []
    for d in range(LATENT_DIM):
        for j in range(_NIDX):
            copies.append(pltpu.async_copy(
                utab_hbm.at[uwidx_v.at[d * _NIDX + j]],
                ubuf.at[d, pl.ds(j * _IDXW, _IDXW)], sem_u))
            copies.append(pltpu.async_copy(
                itab_hbm.at[iwidx_v.at[d * _NIDX + j]],
                ibuf.at[d, pl.ds(j * _IDXW, _IDXW)], sem_i))

    for k in range(HIDDEN):
        lo = w_v[pl.ds(k * LATENT_DIM, LANES)]
        hi = w_v[pl.ds(k * LATENT_DIM + LANES, LANES)]
        for d in range(LANES):
            w_s[k * LATENT_DIM + d] = lo[d]
            w_s[k * LATENT_DIM + LANES + d] = hi[d]
    bvec = b_v[pl.ds(0, LANES)]
    for k in range(HIDDEN):
        b_s[k] = bvec[k]

    for cp in copies:
        cp.wait()

    def group(g, _):
        s0 = g * LANES
        p = []
        q = []
        for k in range(HIDDEN):
            bk = lax.broadcast(b_s[k], (LANES,))
            p.append(bk)
            q.append(bk)
        for d in range(LATENT_DIM):
            ud = ubuf[d, pl.ds(s0, LANES)]
            vd = ibuf[d, pl.ds(s0, LANES)]
            for k in range(HIDDEN):
                w = w_s[k * LATENT_DIM + d]
                p[k] = p[k] + ud * w
                q[k] = q[k] + vd * w
        acc = p[0] * q[0]
        for k in range(1, HIDDEN):
            acc = acc + p[k] * q[k]
        out_v[pl.ds(s0, LANES)] = acc
        return 0

    lax.fori_loop(0, _GROUPS, group, 0)
    pltpu.sync_copy(out_v, out_hbm.at[pl.ds(base, _BPW)])


def kernel(user_ids, item_ids, user_table, item_table, W_user, b_user):
    return _mf_sc(user_ids.astype(jnp.int32), item_ids.astype(jnp.int32),
                  user_table.T.reshape(-1), item_table.T.reshape(-1),
                  W_user.reshape(-1), b_user)


# speedup vs baseline: 5.5208x; 5.5208x over previous
import functools

import jax
import jax.numpy as jnp
from jax import lax
from jax.experimental import pallas as pl
from jax.experimental.pallas import tpu as pltpu
from jax.experimental.pallas import tpu_sc as plsc

BATCH = 16384
LATENT_DIM = 32
HIDDEN = 16
LANES = 16
ROWS_PER_LINE = 4

_info = plsc.get_sparse_core_info()
_NC, _NS = _info.num_cores, _info.num_subcores
_NW = _NC * _NS
_BPW = BATCH // _NW
_WAVE = 256
_WAVES = _BPW // _WAVE
_GROUPS = _WAVE // LANES

_mesh = plsc.VectorSubcoreMesh(core_axis_name="c", subcore_axis_name="s")


@functools.partial(
    pl.kernel,
    mesh=_mesh,
    out_type=jax.ShapeDtypeStruct((BATCH,), jnp.float32),
    compiler_params=pltpu.CompilerParams(needs_layout_passes=False),
    scratch_types=[
        pltpu.VMEM((_BPW,), jnp.int32),
        pltpu.VMEM((_BPW,), jnp.int32),
        pltpu.VMEM((_BPW,), jnp.int32),
        pltpu.VMEM((_BPW,), jnp.int32),
        pltpu.VMEM((_WAVE, 128), jnp.float32),
        pltpu.VMEM((_WAVE, 128), jnp.float32),
        pltpu.VMEM((HIDDEN * LATENT_DIM,), jnp.float32),
        pltpu.VMEM((HIDDEN,), jnp.float32),
        pltpu.SMEM((HIDDEN * LATENT_DIM,), jnp.float32),
        pltpu.SMEM((HIDDEN,), jnp.float32),
        pltpu.VMEM((_BPW,), jnp.float32),
        pltpu.SemaphoreType.DMA,
        pltpu.SemaphoreType.DMA,
    ],
)
def _mf_sc(uids_hbm, iids_hbm, utab_hbm, itab_hbm, w_hbm, b_hbm, out_hbm,
           uidx_v, iidx_v, uline_v, iline_v, ubuf, ibuf, w_v, b_v, w_s, b_s,
           out_v, sem_u, sem_i):
    wid = lax.axis_index("s") * _NC + lax.axis_index("c")
    base = wid * _BPW

    pltpu.sync_copy(uids_hbm.at[pl.ds(base, _BPW)], uidx_v)
    pltpu.sync_copy(iids_hbm.at[pl.ds(base, _BPW)], iidx_v)
    pltpu.sync_copy(w_hbm, w_v)
    pltpu.sync_copy(b_hbm, b_v)

    for t in range(_BPW // LANES):
        sl = pl.ds(t * LANES, LANES)
        uline_v[sl] = lax.shift_right_logical(uidx_v[sl], 2)
        iline_v[sl] = lax.shift_right_logical(iidx_v[sl], 2)

    for k in range(HIDDEN):
        lo = w_v[pl.ds(k * LATENT_DIM, LANES)]
        hi = w_v[pl.ds(k * LATENT_DIM + LANES, LANES)]
        for d in range(LANES):
            w_s[k * LATENT_DIM + d] = lo[d]
            w_s[k * LATENT_DIM + LANES + d] = hi[d]
    bvec = b_v[pl.ds(0, LANES)]
    for k in range(HIDDEN):
        b_s[k] = bvec[k]

    lane = lax.iota(jnp.int32, LANES)

    for wave in range(_WAVES):
        w0 = wave * _WAVE
        cu = pltpu.async_copy(utab_hbm.at[uline_v.at[pl.ds(w0, _WAVE)]],
                              ubuf, sem_u)
        ci = pltpu.async_copy(itab_hbm.at[iline_v.at[pl.ds(w0, _WAVE)]],
                              ibuf, sem_i)
        cu.wait()
        ci.wait()

        def group(g, _):
            s0 = g * LANES
            rows = lane + s0
            ucol = lax.shift_left(
                lax.bitwise_and(uidx_v[pl.ds(w0 + s0, LANES)], 3), 5)
            icol = lax.shift_left(
                lax.bitwise_and(iidx_v[pl.ds(w0 + s0, LANES)], 3), 5)
            p = []
            q = []
            for k in range(HIDDEN):
                bk = lax.broadcast(b_s[k], (LANES,))
                p.append(bk)
                q.append(bk)
            for d in range(LATENT_DIM):
                ud = plsc.load_gather(ubuf, [rows, ucol + d])
                vd = plsc.load_gather(ibuf, [rows, icol + d])
                for k in range(HIDDEN):
                    w = w_s[k * LATENT_DIM + d]
                    p[k] = p[k] + ud * w
                    q[k] = q[k] + vd * w
            acc = p[0] * q[0]
            for k in range(1, HIDDEN):
                acc = acc + p[k] * q[k]
            out_v[pl.ds(w0 + s0, LANES)] = acc
            return 0

        lax.fori_loop(0, _GROUPS, group, 0)

    pltpu.sync_copy(out_v, out_hbm.at[pl.ds(base, _BPW)])


def kernel(user_ids, item_ids, user_table, item_table, W_user, b_user):
    n_lines = user_table.shape[0] // ROWS_PER_LINE
    return _mf_sc(user_ids.astype(jnp.int32), item_ids.astype(jnp.int32),
                  user_table.reshape(n_lines, 128),
                  item_table.reshape(n_lines, 128),
                  W_user.reshape(-1), b_user)


# speedup vs baseline: 5.5406x; 1.0036x over previous
import functools

import jax
import jax.numpy as jnp
from jax import lax
from jax.experimental import pallas as pl
from jax.experimental.pallas import tpu as pltpu
from jax.experimental.pallas import tpu_sc as plsc

BATCH = 16384
LATENT_DIM = 32
HIDDEN = 16
LANES = 16
ROWS_PER_LINE = 4
LINE = ROWS_PER_LINE * LATENT_DIM

_info = plsc.get_sparse_core_info()
_NC, _NS = _info.num_cores, _info.num_subcores
_NW = _NC * _NS
_BPW = BATCH // _NW
_WAVE = 256
_WAVES = _BPW // _WAVE
_GPW = _WAVE // LANES

_mesh = plsc.VectorSubcoreMesh(core_axis_name="c", subcore_axis_name="s")


@functools.partial(
    pl.kernel,
    mesh=_mesh,
    out_type=jax.ShapeDtypeStruct((BATCH,), jnp.float32),
    compiler_params=pltpu.CompilerParams(needs_layout_passes=False),
    scratch_types=[
        pltpu.VMEM((_BPW,), jnp.int32),
        pltpu.VMEM((_BPW,), jnp.int32),
        pltpu.VMEM((_BPW,), jnp.int32),
        pltpu.VMEM((_BPW,), jnp.int32),
        pltpu.VMEM((_WAVE, LINE), jnp.float32),
        pltpu.VMEM((_WAVE, LINE), jnp.float32),
        pltpu.VMEM((HIDDEN * LATENT_DIM,), jnp.float32),
        pltpu.VMEM((HIDDEN,), jnp.float32),
        pltpu.SMEM((HIDDEN * LATENT_DIM,), jnp.float32),
        pltpu.SMEM((HIDDEN,), jnp.float32),
        pltpu.VMEM((_BPW,), jnp.float32),
        pltpu.SemaphoreType.DMA,
        pltpu.SemaphoreType.DMA,
    ],
)
def _mf_sc(uids_hbm, iids_hbm, utab_hbm, itab_hbm, w_hbm, b_hbm, out_hbm,
           uidx_v, iidx_v, uline_v, iline_v, ubuf, ibuf, w_v, b_v, w_s, b_s,
           out_v, sem_u, sem_i):
    wid = lax.axis_index("s") * _NC + lax.axis_index("c")
    base = wid * _BPW

    pltpu.sync_copy(uids_hbm.at[pl.ds(base, _BPW)], uidx_v)
    pltpu.sync_copy(iids_hbm.at[pl.ds(base, _BPW)], iidx_v)
    pltpu.sync_copy(w_hbm, w_v)
    pltpu.sync_copy(b_hbm, b_v)

    for t in range(_BPW // LANES):
        sl = pl.ds(t * LANES, LANES)
        uline_v[sl] = lax.shift_right_logical(uidx_v[sl], 2)
        iline_v[sl] = lax.shift_right_logical(iidx_v[sl], 2)

    for k in range(HIDDEN):
        lo = w_v[pl.ds(k * LATENT_DIM, LANES)]
        hi = w_v[pl.ds(k * LATENT_DIM + LANES, LANES)]
        for d in range(LANES):
            w_s[k * LATENT_DIM + d] = lo[d]
            w_s[k * LATENT_DIM + LANES + d] = hi[d]
    bvec = b_v[pl.ds(0, LANES)]
    for k in range(HIDDEN):
        b_s[k] = bvec[k]

    lane = lax.iota(jnp.int32, LANES)

    def wave(wv, _):
        w0 = wv * _WAVE
        cu = pltpu.async_copy(utab_hbm.at[uline_v.at[pl.ds(w0, _WAVE)]],
                              ubuf, sem_u)
        ci = pltpu.async_copy(itab_hbm.at[iline_v.at[pl.ds(w0, _WAVE)]],
                              ibuf, sem_i)
        cu.wait()
        ci.wait()

        def group(g, _):
            s0 = g * LANES
            slots = lane + s0
            ubase = lax.shift_left(
                lax.bitwise_and(uidx_v[pl.ds(w0 + s0, LANES)], 3), 5)
            ibase = lax.shift_left(
                lax.bitwise_and(iidx_v[pl.ds(w0 + s0, LANES)], 3), 5)
            p = []
            q = []
            for k in range(HIDDEN):
                bk = lax.broadcast(b_s[k], (LANES,))
                p.append(bk)
                q.append(bk)
            for d in range(LATENT_DIM):
                ud = plsc.load_gather(ubuf, [slots, ubase + d])
                vd = plsc.load_gather(ibuf, [slots, ibase + d])
                for k in range(HIDDEN):
                    w = w_s[k * LATENT_DIM + d]
                    p[k] = p[k] + ud * w
                    q[k] = q[k] + vd * w
            acc = p[0] * q[0]
            for k in range(1, HIDDEN):
                acc = acc + p[k] * q[k]
            out_v[pl.ds(w0 + s0, LANES)] = acc
            return 0

        lax.fori_loop(0, _GPW, group, 0)
        return 0

    lax.fori_loop(0, _WAVES, wave, 0)

    pltpu.sync_copy(out_v, out_hbm.at[pl.ds(base, _BPW)])


def kernel(user_ids, item_ids, user_table, item_table, W_user, b_user):
    n_lines = user_table.shape[0] // ROWS_PER_LINE
    ut2 = user_table.reshape(n_lines, LINE)
    it2 = item_table.reshape(n_lines, LINE)
    return _mf_sc(user_ids.astype(jnp.int32), item_ids.astype(jnp.int32),
                  ut2, it2, W_user.reshape(-1), b_user)
